# baseline (speedup 1.0000x reference)
import functools

import jax
import jax.numpy as jnp
from jax import lax
from jax.experimental import pallas as pl
from jax.experimental.pallas import tpu as pltpu
from jax.experimental.pallas import tpu_sc as plsc

_N = 10000
_E = 160000
_DIM = 256
_H = 32
_EPS = 0.5

_NTILES = 16
_EPT = _E // _NTILES
_CH = 80
_NCH = _EPT // _CH
_NPAD = 10240
_RPT = _NPAD // _NTILES


def _mesh():
    return plsc.VectorSubcoreMesh(core_axis_name="c", subcore_axis_name="s")


def _rsqrt16(d):
    u = plsc.bitcast(d, jnp.int32)
    u = jnp.int32(0x5F3759DF) - lax.shift_right_logical(u, 1)
    y = plsc.bitcast(u, jnp.float32)
    for _ in range(3):
        y = y * (1.5 - 0.5 * d * y * y)
    return y


def _sc_msgpass(x, rows_b, cols, al, ar, dv=None):
    fuse_deg = dv is None
    out_types = [jax.ShapeDtypeStruct((2, _NPAD, _H), jnp.float32)]
    scratch = [
        pltpu.VMEM((_NCH, _CH), jnp.int32),
        pltpu.VMEM((_NCH, _CH), jnp.int32),
        pltpu.VMEM((_N,), jnp.float32),
        pltpu.VMEM((_N,), jnp.float32),
        pltpu.VMEM((_NPAD,), jnp.float32),
        pltpu.VMEM((_CH,), jnp.float32),
        pltpu.VMEM((_CH, _H), jnp.float32),
        pltpu.VMEM((_CH, _H), jnp.float32),
        pltpu.VMEM((128, _H), jnp.float32),
        pltpu.VMEM_SHARED((_NPAD, _H), jnp.float32),
        pltpu.SemaphoreType.DMA,
        pltpu.SemaphoreType.DMA,
    ]
    if fuse_deg:
        out_types.append(jax.ShapeDtypeStruct((2, _NPAD), jnp.float32))
        scratch += [
            pltpu.VMEM((_CH, 16), jnp.float32),
            pltpu.VMEM((128, 16), jnp.float32),
            pltpu.VMEM((_RPT, 16), jnp.float32),
            pltpu.VMEM((_RPT,), jnp.float32),
            pltpu.VMEM_SHARED((_NPAD, 16), jnp.float32),
            pltpu.VMEM_SHARED((_NPAD,), jnp.float32),
        ]

    @functools.partial(
        pl.kernel,
        mesh=_mesh(),
        compiler_params=pltpu.CompilerParams(needs_layout_passes=False,
                                             use_tc_tiling_on_sc=False),
        out_type=tuple(out_types) if fuse_deg else out_types[0],
        scratch_types=scratch,
    )
    def k(*refs):
        if fuse_deg:
            (x_h, rows_h, cols_h, al_h, ar_h, acc_h, dvout_h,
             rows_v, cols_v, al_v, ar_v, dv_v, cbuf, xg0, xg1, zb_v, acc_s,
             semg0, semg1, ones_v, zb16_v, db_v, dvl_v, deg_s, dv_sp) = refs
        else:
            (x_h, rows_h, cols_h, al_h, ar_h, dv_h, acc_h,
             rows_v, cols_v, al_v, ar_v, dv_v, cbuf, xg0, xg1, zb_v, acc_s,
             semg0, semg1) = refs
        c = lax.axis_index("c")
        s = lax.axis_index("s")
        w = c * _NTILES + s
        base = s * _RPT
        pltpu.sync_copy(rows_h.at[w], rows_v)
        pltpu.sync_copy(cols_h.at[w], cols_v)
        pltpu.sync_copy(al_h.at[c], al_v)
        pltpu.sync_copy(ar_h.at[c], ar_v)
        zero16 = jnp.zeros((16,), jnp.float32)
        for r in range(128):
            zb_v[r, pl.ds(0, 16)] = zero16
            zb_v[r, pl.ds(16, 16)] = zero16
        for i in range(5):
            pltpu.sync_copy(zb_v, acc_s.at[pl.ds(base + i * 128, 128)])

        if fuse_deg:
            one16 = jnp.ones((16,), jnp.float32)
            for r in range(_CH):
                ones_v[r, pl.ds(0, 16)] = one16
            for r in range(128):
                zb16_v[r, pl.ds(0, 16)] = zero16
            for i in range(5):
                pltpu.sync_copy(zb16_v, deg_s.at[pl.ds(base + i * 128, 128)])
            plsc.subcore_barrier()

            def degchunk(j, carry):
                pltpu.sync_copy(ones_v, deg_s.at[cols_v.at[j]], add=True)
                return carry

            lax.fori_loop(0, _NCH, degchunk, 0)
            plsc.subcore_barrier()
            pltpu.sync_copy(deg_s.at[pl.ds(base, _RPT)], db_v)
            lane = jnp.arange(16, dtype=jnp.int32)
            zl = jnp.zeros((16,), jnp.int32)
            for g in range(_RPT // 16):
                d = plsc.load_gather(db_v, [lane + g * 16, zl])
                dvl_v[pl.ds(g * 16, 16)] = _rsqrt16(d + 1.0)
            pltpu.sync_copy(dvl_v, dv_sp.at[pl.ds(base, _RPT)])
            pltpu.sync_copy(dvl_v, dvout_h.at[c, pl.ds(base, _RPT)])
            plsc.subcore_barrier()
            pltpu.sync_copy(dv_sp, dv_v)
        else:
            pltpu.sync_copy(dv_h.at[c], dv_v)
            plsc.subcore_barrier()

        coff = c * _N

        def coeffs(j):
            for kk in range(_CH // 16):
                rb = rows_v[j, pl.ds(kk * 16, 16)]
                rl = rb - coff
                cl = cols_v[j, pl.ds(kk * 16, 16)]
                alr = plsc.load_gather(al_v, [rl])
                arc = plsc.load_gather(ar_v, [cl])
                dvr = plsc.load_gather(dv_v, [rl])
                dvc = plsc.load_gather(dv_v, [cl])
                a = alr + arc
                e2 = jnp.exp(a + a)
                th = 1.0 - 2.0 / (e2 + 1.0)
                cbuf[pl.ds(kk * 16, 16)] = th * dvr * dvc

        def scale(xg):
            for g in range(_CH // 16):
                cg = cbuf[pl.ds(g * 16, 16)]
                for lane in range(16):
                    e = g * 16 + lane
                    cv = jnp.full((16,), cg[lane], jnp.float32)
                    xg[e, pl.ds(0, 16)] = xg[e, pl.ds(0, 16)] * cv
                    xg[e, pl.ds(16, 16)] = xg[e, pl.ds(16, 16)] * cv

        def issue_gather(j, xg, semg):
            pltpu.async_copy(x_h.at[rows_v.at[j]], xg, semg)

        def wait_gather(j, xg, semg):
            pltpu.make_async_copy(x_h.at[rows_v.at[j]], xg, semg).wait()

        def issue_scatter(j, xg, sems):
            pltpu.async_copy(xg, acc_s.at[cols_v.at[j]], sems, add=True)

        def wait_scatter(j, xg, sems):
            pltpu.make_async_copy(xg, acc_s.at[cols_v.at[j]], sems).wait()

        issue_gather(0, xg0, semg0)
        coeffs(0)
        wait_gather(0, xg0, semg0)
        issue_gather(1, xg1, semg1)
        scale(xg0)
        pltpu.sync_copy(xg0, acc_s.at[cols_v.at[0]], add=True)

        def pair(i, carry):
            ja = 2 * i + 1
            jb = 2 * i + 2
            coeffs(ja)
            wait_gather(ja, xg1, semg1)
            issue_gather(jb, xg0, semg0)
            scale(xg1)
            pltpu.sync_copy(xg1, acc_s.at[cols_v.at[ja]], add=True)
            coeffs(jb)
            wait_gather(jb, xg0, semg0)

            @pl.when(jb + 1 < _NCH)
            def _():
                issue_gather(jb + 1, xg1, semg1)

            scale(xg0)
            pltpu.sync_copy(xg0, acc_s.at[cols_v.at[jb]], add=True)
            return carry

        lax.fori_loop(0, (_NCH - 1) // 2, pair, 0)
        plsc.subcore_barrier()
        pltpu.sync_copy(acc_s.at[pl.ds(base, _RPT)],
                        acc_h.at[c, pl.ds(base, _RPT)])

    if fuse_deg:
        return k(x, rows_b, cols, al, ar)
    return k(x, rows_b, cols, al, ar, dv)


def _softplus(z):
    return jnp.maximum(z, 0.0) + jnp.log1p(jnp.exp(-jnp.abs(z)))


def _tc_stage_a(X, W_shr, b_shr, W_src, b_src, W_trg, b_trg,
                Wd1, bd1, Wd2, bd2, att_l, att_r):
    BLK = 1000
    G = (2 * _N) // BLK
    half = G // 2

    def body(x_ref, wsh_ref, bsh_ref, wsr_ref, bsr_ref, wtg_ref,
             btg_ref, wd1_ref, bd1_ref, wd2_ref, bd2_ref, atl_ref, atr_ref,
             h_ref, al_ref, ar_ref, disc_ref):
        i = pl.program_id(0)
        x = x_ref[...]
        shr = jnp.maximum(
            jnp.dot(x, wsh_ref[...], preferred_element_type=jnp.float32)
            + bsh_ref[...], 0.0)
        wown = jnp.where(i < half, wsr_ref[...], wtg_ref[...])
        bown = jnp.where(i < half, bsr_ref[...], btg_ref[...])
        h = jnp.maximum(
            jnp.dot(x, wown, preferred_element_type=jnp.float32) + bown,
            0.0) + shr

        def disc(m):
            z1 = jnp.maximum(
                jnp.dot(m, wd1_ref[...], preferred_element_type=jnp.float32)
                + bd1_ref[...], 0.0)
            return (jnp.dot(z1, wd2_ref[...],
                            preferred_element_type=jnp.float32)
                    + bd2_ref[...])

        sgn = jnp.where(i < half, 1.0, -1.0)
        part = (jnp.sum(_softplus(sgn * disc(shr)))
                + jnp.sum(_softplus(sgn * disc(h))))

        @pl.when(i == 0)
        def _():
            disc_ref[...] = jnp.zeros((1, 1), jnp.float32)

        disc_ref[...] += jnp.reshape(part, (1, 1))
        h_ref[...] = h
        al_ref[...] = jnp.dot(h, atl_ref[...],
                              preferred_element_type=jnp.float32)
        ar_ref[...] = jnp.dot(h, atr_ref[...],
                              preferred_element_type=jnp.float32)

    full = lambda shape: pl.BlockSpec(shape, lambda i: (0, 0))
    return pl.pallas_call(
        body,
        grid=(G,),
        in_specs=[
            pl.BlockSpec((BLK, _DIM), lambda i: (i, 0)),
            full((_DIM, _H)), full((1, _H)),
            full((_DIM, _H)), full((1, _H)),
            full((_DIM, _H)), full((1, _H)),
            full((_H, 16)), full((1, 16)),
            full((16, 1)), full((1, 1)),
            full((_H, 1)), full((_H, 1)),
        ],
        out_specs=[
            pl.BlockSpec((BLK, _H), lambda i: (i, 0)),
            pl.BlockSpec((BLK, 1), lambda i: (i, 0)),
            pl.BlockSpec((BLK, 1), lambda i: (i, 0)),
            pl.BlockSpec((1, 1), lambda i: (0, 0)),
        ],
        out_shape=[
            jax.ShapeDtypeStruct((2 * _N, _H), jnp.float32),
            jax.ShapeDtypeStruct((2 * _N, 1), jnp.float32),
            jax.ShapeDtypeStruct((2 * _N, 1), jnp.float32),
            jax.ShapeDtypeStruct((1, 1), jnp.float32),
        ],
    )(X, W_shr, b_shr, W_src, b_src, W_trg, b_trg,
      Wd1, bd1, Wd2, bd2, att_l, att_r)


def _tc_stage_c(acc, h, al, ar, dinv, att_l, att_r):
    BLK = 2000
    G = (2 * _N) // BLK

    def body(acc_ref, h_ref, al_ref, ar_ref, dv_ref, atl_ref, atr_ref,
             x1_ref, al2_ref, ar2_ref):
        dv = dv_ref[...]
        coef = jnp.tanh(al_ref[...] + ar_ref[...]) * dv * dv + _EPS
        x1 = acc_ref[...] + coef * h_ref[...]
        x1_ref[...] = x1
        al2_ref[...] = jnp.dot(x1, atl_ref[...],
                               preferred_element_type=jnp.float32)
        ar2_ref[...] = jnp.dot(x1, atr_ref[...],
                               preferred_element_type=jnp.float32)

    col = lambda: pl.BlockSpec((BLK, 1), lambda i: (i, 0))
    return pl.pallas_call(
        body,
        grid=(G,),
        in_specs=[
            pl.BlockSpec((BLK, _H), lambda i: (i, 0)),
            pl.BlockSpec((BLK, _H), lambda i: (i, 0)),
            col(), col(), col(),
            pl.BlockSpec((_H, 1), lambda i: (0, 0)),
            pl.BlockSpec((_H, 1), lambda i: (0, 0)),
        ],
        out_specs=[
            pl.BlockSpec((BLK, _H), lambda i: (i, 0)),
            col(), col(),
        ],
        out_shape=[
            jax.ShapeDtypeStruct((2 * _N, _H), jnp.float32),
            jax.ShapeDtypeStruct((2 * _N, 1), jnp.float32),
            jax.ShapeDtypeStruct((2 * _N, 1), jnp.float32),
        ],
    )(acc, h, al, ar, dinv, att_l, att_r)


def _tc_stage_d(acc2, x1, h, al2, ar2, dinv, W_pred, b_pred):
    BLK = 2000
    G = (2 * _N) // BLK

    def body(acc_ref, x1_ref, h_ref, al_ref, ar_ref, dv_ref, wp_ref, bp_ref,
             out_ref):
        dv = dv_ref[...]
        coef = jnp.tanh(al_ref[...] + ar_ref[...]) * dv * dv
        y = acc_ref[...] + coef * x1_ref[...] + _EPS * h_ref[...]
        out_ref[...] = (jnp.dot(y, wp_ref[...],
                                preferred_element_type=jnp.float32)
                        + bp_ref[...])

    col = lambda: pl.BlockSpec((BLK, 1), lambda i: (i, 0))
    return pl.pallas_call(
        body,
        grid=(G,),
        in_specs=[
            pl.BlockSpec((BLK, _H), lambda i: (i, 0)),
            pl.BlockSpec((BLK, _H), lambda i: (i, 0)),
            pl.BlockSpec((BLK, _H), lambda i: (i, 0)),
            col(), col(), col(),
            pl.BlockSpec((_H, 2), lambda i: (0, 0)),
            pl.BlockSpec((1, 2), lambda i: (0, 0)),
        ],
        out_specs=pl.BlockSpec((BLK, 2), lambda i: (i, 0)),
        out_shape=jax.ShapeDtypeStruct((2 * _N, 2), jnp.float32),
    )(acc2, x1, h, al2, ar2, dinv, W_pred, b_pred)


def kernel(src, src_edge, trg, trg_edge, W_src, b_src, W_shr, b_shr,
           W_trg, b_trg, Wd1, bd1, Wd2, bd2, att_l, att_r, W_pred, b_pred):
    e_s = src_edge[0]
    e_t = trg_edge[0]
    rows_b = jnp.concatenate(
        [e_s[0], e_t[0] + _N]).reshape(32, _NCH, _CH)
    cols = jnp.concatenate([e_s[1], e_t[1]]).reshape(32, _NCH, _CH)

    X = jnp.concatenate([src, trg], axis=0)

    h, al1, ar1, disc_sum = _tc_stage_a(
        X, W_shr, b_shr.reshape(1, _H),
        W_src, b_src.reshape(1, _H), W_trg, b_trg.reshape(1, _H),
        Wd1, bd1.reshape(1, 16), Wd2, bd2.reshape(1, 1), att_l, att_r)

    al1_t = al1.reshape(2, _N)
    ar1_t = ar1.reshape(2, _N)

    acc1, dvp = _sc_msgpass(h, rows_b, cols, al1_t, ar1_t)
    dinv = dvp[:, :_N].reshape(2 * _N, 1)
    x1, al2, ar2 = _tc_stage_c(acc1[:, :_N].reshape(2 * _N, _H), h, al1,
                               ar1, dinv, att_l, att_r)
    acc2 = _sc_msgpass(x1, rows_b, cols, al2.reshape(2, _N),
                       ar2.reshape(2, _N), dvp)
    out = _tc_stage_d(acc2[:, :_N].reshape(2 * _N, _H), x1, h, al2, ar2,
                      dinv, W_pred, b_pred.reshape(1, 2))

    disc_loss = disc_sum[0, 0] / jnp.float32(_N)
    return out[:_N], out[_N:], disc_loss


# speedup vs baseline: 1.0134x; 1.0134x over previous
import functools

import jax
import jax.numpy as jnp
from jax import lax
from jax.experimental import pallas as pl
from jax.experimental.pallas import tpu as pltpu
from jax.experimental.pallas import tpu_sc as plsc

_N = 10000
_E = 160000
_DIM = 256
_H = 32
_EPS = 0.5

_NTILES = 16
_EPT = _E // _NTILES
_CH = 80
_NCH = _EPT // _CH
_NPAD = 10240
_RPT = _NPAD // _NTILES


def _mesh():
    return plsc.VectorSubcoreMesh(core_axis_name="c", subcore_axis_name="s")


def _sc_degree(cols):
    @functools.partial(
        pl.kernel,
        mesh=_mesh(),
        out_type=jax.ShapeDtypeStruct((2, _NPAD, 16), jnp.float32),
        scratch_types=[
            pltpu.VMEM((_NCH, _CH), jnp.int32),
            pltpu.VMEM((_CH, 16), jnp.float32),
            pltpu.VMEM((128, 16), jnp.float32),
            pltpu.VMEM_SHARED((_NPAD, 16), jnp.float32),
        ],
    )
    def k(cols_h, deg_h, cols_v, ones_v, zb_v, deg_s):
        c = lax.axis_index("c")
        s = lax.axis_index("s")
        w = c * _NTILES + s
        pltpu.sync_copy(cols_h.at[w], cols_v)
        one16 = jnp.ones((16,), jnp.float32)
        zero16 = jnp.zeros((16,), jnp.float32)
        for r in range(_CH):
            ones_v[r, pl.ds(0, 16)] = one16
        for r in range(128):
            zb_v[r, pl.ds(0, 16)] = zero16
        base = s * _RPT
        for i in range(5):
            pltpu.sync_copy(zb_v, deg_s.at[pl.ds(base + i * 128, 128)])
        plsc.subcore_barrier()

        def chunk(j, carry):
            pltpu.sync_copy(ones_v, deg_s.at[cols_v.at[j]], add=True)
            return carry

        lax.fori_loop(0, _NCH, chunk, 0)
        plsc.subcore_barrier()
        pltpu.sync_copy(deg_s.at[pl.ds(base, _RPT)],
                        deg_h.at[c, pl.ds(base, _RPT)])

    return k(cols)


def _sc_msgpass(x, rows_b, cols, al, ar, dv):
    @functools.partial(
        pl.kernel,
        mesh=_mesh(),
        compiler_params=pltpu.CompilerParams(needs_layout_passes=False,
                                             use_tc_tiling_on_sc=False),
        out_type=jax.ShapeDtypeStruct((2, _NPAD, _H), jnp.float32),
        scratch_types=[
            pltpu.VMEM((_NCH, _CH), jnp.int32),
            pltpu.VMEM((_NCH, _CH), jnp.int32),
            pltpu.VMEM((_N,), jnp.float32),
            pltpu.VMEM((_N,), jnp.float32),
            pltpu.VMEM((_N,), jnp.float32),
            pltpu.VMEM((_CH,), jnp.float32),
            pltpu.VMEM((_CH, _H), jnp.float32),
            pltpu.VMEM((_CH, _H), jnp.float32),
            pltpu.VMEM((128, _H), jnp.float32),
            pltpu.VMEM_SHARED((_NPAD, _H), jnp.float32),
            pltpu.SemaphoreType.DMA,
            pltpu.SemaphoreType.DMA,
            pltpu.SemaphoreType.DMA,
            pltpu.SemaphoreType.DMA,
        ],
    )
    def k(x_h, rows_h, cols_h, al_h, ar_h, dv_h, acc_h,
          rows_v, cols_v, al_v, ar_v, dv_v, cbuf, xg0, xg1, zb_v, acc_s,
          semg0, semg1, sems0, sems1):
        c = lax.axis_index("c")
        s = lax.axis_index("s")
        w = c * _NTILES + s
        pltpu.sync_copy(rows_h.at[w], rows_v)
        pltpu.sync_copy(cols_h.at[w], cols_v)
        pltpu.sync_copy(al_h.at[c], al_v)
        pltpu.sync_copy(ar_h.at[c], ar_v)
        pltpu.sync_copy(dv_h.at[c], dv_v)
        zero16 = jnp.zeros((16,), jnp.float32)
        for r in range(128):
            zb_v[r, pl.ds(0, 16)] = zero16
            zb_v[r, pl.ds(16, 16)] = zero16
        base = s * _RPT
        for i in range(5):
            pltpu.sync_copy(zb_v, acc_s.at[pl.ds(base + i * 128, 128)])
        plsc.subcore_barrier()

        coff = c * _N

        def coeffs(j):
            for kk in range(_CH // 16):
                rb = rows_v[j, pl.ds(kk * 16, 16)]
                rl = rb - coff
                cl = cols_v[j, pl.ds(kk * 16, 16)]
                alr = plsc.load_gather(al_v, [rl])
                arc = plsc.load_gather(ar_v, [cl])
                dvr = plsc.load_gather(dv_v, [rl])
                dvc = plsc.load_gather(dv_v, [cl])
                a = alr + arc
                e2 = jnp.exp(a + a)
                th = 1.0 - 2.0 / (e2 + 1.0)
                cbuf[pl.ds(kk * 16, 16)] = th * dvr * dvc

        def scale(xg):
            for g in range(_CH // 16):
                cg = cbuf[pl.ds(g * 16, 16)]
                for lane in range(16):
                    e = g * 16 + lane
                    cv = jnp.full((16,), cg[lane], jnp.float32)
                    xg[e, pl.ds(0, 16)] = xg[e, pl.ds(0, 16)] * cv
                    xg[e, pl.ds(16, 16)] = xg[e, pl.ds(16, 16)] * cv

        def issue_gather(j, xg, semg):
            pltpu.async_copy(x_h.at[rows_v.at[j]], xg, semg)

        def wait_gather(j, xg, semg):
            pltpu.make_async_copy(x_h.at[rows_v.at[j]], xg, semg).wait()

        def issue_scatter(j, xg, sems):
            pltpu.async_copy(xg, acc_s.at[cols_v.at[j]], sems, add=True)

        def wait_scatter(j, xg, sems):
            pltpu.make_async_copy(xg, acc_s.at[cols_v.at[j]], sems).wait()

        issue_gather(0, xg0, semg0)
        coeffs(0)
        wait_gather(0, xg0, semg0)
        issue_gather(1, xg1, semg1)
        scale(xg0)
        pltpu.sync_copy(xg0, acc_s.at[cols_v.at[0]], add=True)

        def pair(i, carry):
            ja = 2 * i + 1
            jb = 2 * i + 2
            coeffs(ja)
            wait_gather(ja, xg1, semg1)
            issue_gather(jb, xg0, semg0)
            scale(xg1)
            pltpu.sync_copy(xg1, acc_s.at[cols_v.at[ja]], add=True)
            coeffs(jb)
            wait_gather(jb, xg0, semg0)

            @pl.when(jb + 1 < _NCH)
            def _():
                issue_gather(jb + 1, xg1, semg1)

            scale(xg0)
            pltpu.sync_copy(xg0, acc_s.at[cols_v.at[jb]], add=True)
            return carry

        lax.fori_loop(0, (_NCH - 1) // 2, pair, 0)
        plsc.subcore_barrier()
        pltpu.sync_copy(acc_s.at[pl.ds(base, _RPT)],
                        acc_h.at[c, pl.ds(base, _RPT)])

    return k(x, rows_b, cols, al, ar, dv)


def _softplus(z):
    return jnp.maximum(z, 0.0) + jnp.log1p(jnp.exp(-jnp.abs(z)))


def _tc_stage_a(X, deg, W_shr, b_shr, W_src, b_src, W_trg, b_trg,
                Wd1, bd1, Wd2, bd2, att_l, att_r):
    BLK = 1000
    G = (2 * _N) // BLK
    half = G // 2

    def body(x_ref, deg_ref, wsh_ref, bsh_ref, wsr_ref, bsr_ref, wtg_ref,
             btg_ref, wd1_ref, bd1_ref, wd2_ref, bd2_ref, atl_ref, atr_ref,
             h_ref, al_ref, ar_ref, dinv_ref, disc_ref):
        i = pl.program_id(0)
        x = x_ref[...]
        shr = jnp.maximum(
            jnp.dot(x, wsh_ref[...], preferred_element_type=jnp.float32)
            + bsh_ref[...], 0.0)
        wown = jnp.where(i < half, wsr_ref[...], wtg_ref[...])
        bown = jnp.where(i < half, bsr_ref[...], btg_ref[...])
        h = jnp.maximum(
            jnp.dot(x, wown, preferred_element_type=jnp.float32) + bown,
            0.0) + shr

        def disc(m):
            z1 = jnp.maximum(
                jnp.dot(m, wd1_ref[...], preferred_element_type=jnp.float32)
                + bd1_ref[...], 0.0)
            return (jnp.dot(z1, wd2_ref[...],
                            preferred_element_type=jnp.float32)
                    + bd2_ref[...])

        sgn = jnp.where(i < half, 1.0, -1.0)
        part = (jnp.sum(_softplus(sgn * disc(shr)))
                + jnp.sum(_softplus(sgn * disc(h))))

        @pl.when(i == 0)
        def _():
            disc_ref[...] = jnp.zeros((1, 1), jnp.float32)

        disc_ref[...] += jnp.reshape(part, (1, 1))
        h_ref[...] = h
        al_ref[...] = jnp.dot(h, atl_ref[...],
                              preferred_element_type=jnp.float32)
        ar_ref[...] = jnp.dot(h, atr_ref[...],
                              preferred_element_type=jnp.float32)
        dinv_ref[...] = lax.rsqrt(deg_ref[:, 0:1] + 1.0)

    full = lambda shape: pl.BlockSpec(shape, lambda i: (0, 0))
    return pl.pallas_call(
        body,
        grid=(G,),
        in_specs=[
            pl.BlockSpec((BLK, _DIM), lambda i: (i, 0)),
            pl.BlockSpec((BLK, 16), lambda i: (i, 0)),
            full((_DIM, _H)), full((1, _H)),
            full((_DIM, _H)), full((1, _H)),
            full((_DIM, _H)), full((1, _H)),
            full((_H, 16)), full((1, 16)),
            full((16, 1)), full((1, 1)),
            full((_H, 1)), full((_H, 1)),
        ],
        out_specs=[
            pl.BlockSpec((BLK, _H), lambda i: (i, 0)),
            pl.BlockSpec((BLK, 1), lambda i: (i, 0)),
            pl.BlockSpec((BLK, 1), lambda i: (i, 0)),
            pl.BlockSpec((BLK, 1), lambda i: (i, 0)),
            pl.BlockSpec((1, 1), lambda i: (0, 0)),
        ],
        out_shape=[
            jax.ShapeDtypeStruct((2 * _N, _H), jnp.float32),
            jax.ShapeDtypeStruct((2 * _N, 1), jnp.float32),
            jax.ShapeDtypeStruct((2 * _N, 1), jnp.float32),
            jax.ShapeDtypeStruct((2 * _N, 1), jnp.float32),
            jax.ShapeDtypeStruct((1, 1), jnp.float32),
        ],
    )(X, deg, W_shr, b_shr, W_src, b_src, W_trg, b_trg,
      Wd1, bd1, Wd2, bd2, att_l, att_r)


def _tc_stage_c(acc, h, al, ar, dinv, att_l, att_r):
    BLK = 2000
    G = (2 * _N) // BLK

    def body(acc_ref, h_ref, al_ref, ar_ref, dv_ref, atl_ref, atr_ref,
             x1_ref, al2_ref, ar2_ref):
        dv = dv_ref[...]
        coef = jnp.tanh(al_ref[...] + ar_ref[...]) * dv * dv + _EPS
        x1 = acc_ref[...] + coef * h_ref[...]
        x1_ref[...] = x1
        al2_ref[...] = jnp.dot(x1, atl_ref[...],
                               preferred_element_type=jnp.float32)
        ar2_ref[...] = jnp.dot(x1, atr_ref[...],
                               preferred_element_type=jnp.float32)

    col = lambda: pl.BlockSpec((BLK, 1), lambda i: (i, 0))
    return pl.pallas_call(
        body,
        grid=(G,),
        in_specs=[
            pl.BlockSpec((BLK, _H), lambda i: (i, 0)),
            pl.BlockSpec((BLK, _H), lambda i: (i, 0)),
            col(), col(), col(),
            pl.BlockSpec((_H, 1), lambda i: (0, 0)),
            pl.BlockSpec((_H, 1), lambda i: (0, 0)),
        ],
        out_specs=[
            pl.BlockSpec((BLK, _H), lambda i: (i, 0)),
            col(), col(),
        ],
        out_shape=[
            jax.ShapeDtypeStruct((2 * _N, _H), jnp.float32),
            jax.ShapeDtypeStruct((2 * _N, 1), jnp.float32),
            jax.ShapeDtypeStruct((2 * _N, 1), jnp.float32),
        ],
    )(acc, h, al, ar, dinv, att_l, att_r)


def _tc_stage_d(acc2, x1, h, al2, ar2, dinv, W_pred, b_pred):
    BLK = 2000
    G = (2 * _N) // BLK

    def body(acc_ref, x1_ref, h_ref, al_ref, ar_ref, dv_ref, wp_ref, bp_ref,
             out_ref):
        dv = dv_ref[...]
        coef = jnp.tanh(al_ref[...] + ar_ref[...]) * dv * dv
        y = acc_ref[...] + coef * x1_ref[...] + _EPS * h_ref[...]
        out_ref[...] = (jnp.dot(y, wp_ref[...],
                                preferred_element_type=jnp.float32)
                        + bp_ref[...])

    col = lambda: pl.BlockSpec((BLK, 1), lambda i: (i, 0))
    return pl.pallas_call(
        body,
        grid=(G,),
        in_specs=[
            pl.BlockSpec((BLK, _H), lambda i: (i, 0)),
            pl.BlockSpec((BLK, _H), lambda i: (i, 0)),
            pl.BlockSpec((BLK, _H), lambda i: (i, 0)),
            col(), col(), col(),
            pl.BlockSpec((_H, 2), lambda i: (0, 0)),
            pl.BlockSpec((1, 2), lambda i: (0, 0)),
        ],
        out_specs=pl.BlockSpec((BLK, 2), lambda i: (i, 0)),
        out_shape=jax.ShapeDtypeStruct((2 * _N, 2), jnp.float32),
    )(acc2, x1, h, al2, ar2, dinv, W_pred, b_pred)


def kernel(src, src_edge, trg, trg_edge, W_src, b_src, W_shr, b_shr,
           W_trg, b_trg, Wd1, bd1, Wd2, bd2, att_l, att_r, W_pred, b_pred):
    e_s = src_edge[0]
    e_t = trg_edge[0]
    rows_b = jnp.concatenate(
        [e_s[0], e_t[0] + _N]).reshape(32, _NCH, _CH)
    cols = jnp.concatenate([e_s[1], e_t[1]]).reshape(32, _NCH, _CH)

    X = jnp.concatenate([src, trg], axis=0)
    deg = _sc_degree(cols)

    h, al1, ar1, dinv, disc_sum = _tc_stage_a(
        X, deg[:, :_N].reshape(2 * _N, 16), W_shr, b_shr.reshape(1, _H),
        W_src, b_src.reshape(1, _H), W_trg, b_trg.reshape(1, _H),
        Wd1, bd1.reshape(1, 16), Wd2, bd2.reshape(1, 1), att_l, att_r)

    al1_t = al1.reshape(2, _N)
    ar1_t = ar1.reshape(2, _N)
    dv_t = dinv.reshape(2, _N)

    acc1 = _sc_msgpass(h, rows_b, cols, al1_t, ar1_t, dv_t)
    x1, al2, ar2 = _tc_stage_c(acc1[:, :_N].reshape(2 * _N, _H), h, al1,
                               ar1, dinv, att_l, att_r)
    acc2 = _sc_msgpass(x1, rows_b, cols, al2.reshape(2, _N),
                       ar2.reshape(2, _N), dv_t)
    out = _tc_stage_d(acc2[:, :_N].reshape(2 * _N, _H), x1, h, al2, ar2,
                      dinv, W_pred, b_pred.reshape(1, 2))

    disc_loss = disc_sum[0, 0] / jnp.float32(_N)
    return out[:_N], out[_N:], disc_loss


# speedup vs baseline: 1.0462x; 1.0323x over previous
import functools

import jax
import jax.numpy as jnp
from jax import lax
from jax.experimental import pallas as pl
from jax.experimental.pallas import tpu as pltpu
from jax.experimental.pallas import tpu_sc as plsc

_N = 10000
_E = 160000
_DIM = 256
_H = 32
_EPS = 0.5

_NTILES = 16
_EPT = _E // _NTILES
_CH = 80
_NCH = _EPT // _CH
_NPAD = 10240
_RPT = _NPAD // _NTILES


def _mesh():
    return plsc.VectorSubcoreMesh(core_axis_name="c", subcore_axis_name="s")


def _sc_degree(cols):
    @functools.partial(
        pl.kernel,
        mesh=_mesh(),
        out_type=jax.ShapeDtypeStruct((2, _NPAD, 16), jnp.float32),
        scratch_types=[
            pltpu.VMEM((_NCH, _CH), jnp.int32),
            pltpu.VMEM((_CH, 16), jnp.float32),
            pltpu.VMEM((128, 16), jnp.float32),
            pltpu.VMEM_SHARED((_NPAD, 16), jnp.float32),
        ],
    )
    def k(cols_h, deg_h, cols_v, ones_v, zb_v, deg_s):
        c = lax.axis_index("c")
        s = lax.axis_index("s")
        w = c * _NTILES + s
        pltpu.sync_copy(cols_h.at[w], cols_v)
        one16 = jnp.ones((16,), jnp.float32)
        zero16 = jnp.zeros((16,), jnp.float32)
        for r in range(_CH):
            ones_v[r, pl.ds(0, 16)] = one16
        for r in range(128):
            zb_v[r, pl.ds(0, 16)] = zero16
        base = s * _RPT
        for i in range(5):
            pltpu.sync_copy(zb_v, deg_s.at[pl.ds(base + i * 128, 128)])
        plsc.subcore_barrier()

        def chunk(j, carry):
            pltpu.sync_copy(ones_v, deg_s.at[cols_v.at[j]], add=True)
            return carry

        lax.fori_loop(0, _NCH, chunk, 0)
        plsc.subcore_barrier()
        pltpu.sync_copy(deg_s.at[pl.ds(base, _RPT)],
                        deg_h.at[c, pl.ds(base, _RPT)])

    return k(cols)


def _sc_msgpass(x, rows_b, cols, al, ar, dv):
    @functools.partial(
        pl.kernel,
        mesh=_mesh(),
        compiler_params=pltpu.CompilerParams(needs_layout_passes=False,
                                             use_tc_tiling_on_sc=False),
        out_type=jax.ShapeDtypeStruct((2, _NPAD, _H), jnp.float32),
        scratch_types=[
            pltpu.VMEM((_NCH, _CH), jnp.int32),
            pltpu.VMEM((_NCH, _CH), jnp.int32),
            pltpu.VMEM((_N,), jnp.float32),
            pltpu.VMEM((_N,), jnp.float32),
            pltpu.VMEM((_N,), jnp.float32),
            pltpu.VMEM((_CH,), jnp.float32),
            pltpu.VMEM((_CH, _H), jnp.float32),
            pltpu.VMEM((_CH, _H), jnp.float32),
            pltpu.VMEM((128, _H), jnp.float32),
            pltpu.VMEM_SHARED((_NPAD, _H), jnp.float32),
            pltpu.SemaphoreType.DMA,
            pltpu.SemaphoreType.DMA,
            pltpu.SemaphoreType.DMA,
            pltpu.SemaphoreType.DMA,
        ],
    )
    def k(x_h, rows_h, cols_h, al_h, ar_h, dv_h, acc_h,
          rows_v, cols_v, al_v, ar_v, dv_v, cbuf, xg0, xg1, zb_v, acc_s,
          semg0, semg1, sems0, sems1):
        c = lax.axis_index("c")
        s = lax.axis_index("s")
        w = c * _NTILES + s
        pltpu.sync_copy(rows_h.at[w], rows_v)
        pltpu.sync_copy(cols_h.at[w], cols_v)
        pltpu.sync_copy(al_h.at[c], al_v)
        pltpu.sync_copy(ar_h.at[c], ar_v)
        pltpu.sync_copy(dv_h.at[c], dv_v)
        zero16 = jnp.zeros((16,), jnp.float32)
        for r in range(128):
            zb_v[r, pl.ds(0, 16)] = zero16
            zb_v[r, pl.ds(16, 16)] = zero16
        base = s * _RPT
        for i in range(5):
            pltpu.sync_copy(zb_v, acc_s.at[pl.ds(base + i * 128, 128)])
        plsc.subcore_barrier()

        coff = c * _N

        def coeffs(j):
            for kk in range(_CH // 16):
                rb = rows_v[j, pl.ds(kk * 16, 16)]
                rl = rb - coff
                cl = cols_v[j, pl.ds(kk * 16, 16)]
                alr = plsc.load_gather(al_v, [rl])
                arc = plsc.load_gather(ar_v, [cl])
                dvr = plsc.load_gather(dv_v, [rl])
                dvc = plsc.load_gather(dv_v, [cl])
                a = alr + arc
                e2 = jnp.exp(a + a)
                th = 1.0 - 2.0 / (e2 + 1.0)
                cbuf[pl.ds(kk * 16, 16)] = th * dvr * dvc

        def scale(xg):
            for g in range(_CH // 16):
                cg = cbuf[pl.ds(g * 16, 16)]
                for lane in range(16):
                    e = g * 16 + lane
                    cv = jnp.full((16,), cg[lane], jnp.float32)
                    xg[e, pl.ds(0, 16)] = xg[e, pl.ds(0, 16)] * cv
                    xg[e, pl.ds(16, 16)] = xg[e, pl.ds(16, 16)] * cv

        def issue_gather(j, xg, semg):
            pltpu.async_copy(x_h.at[rows_v.at[j]], xg, semg)

        def wait_gather(j, xg, semg):
            pltpu.make_async_copy(x_h.at[rows_v.at[j]], xg, semg).wait()

        def issue_scatter(j, xg, sems):
            pltpu.async_copy(xg, acc_s.at[cols_v.at[j]], sems, add=True)

        def wait_scatter(j, xg, sems):
            pltpu.make_async_copy(xg, acc_s.at[cols_v.at[j]], sems).wait()

        issue_gather(0, xg0, semg0)
        coeffs(0)
        wait_gather(0, xg0, semg0)
        issue_gather(1, xg1, semg1)
        scale(xg0)
        pltpu.sync_copy(xg0, acc_s.at[cols_v.at[0]], add=True)

        def pair(i, carry):
            ja = 2 * i + 1
            jb = 2 * i + 2
            coeffs(ja)
            wait_gather(ja, xg1, semg1)
            issue_gather(jb, xg0, semg0)
            scale(xg1)
            pltpu.sync_copy(xg1, acc_s.at[cols_v.at[ja]], add=True)
            coeffs(jb)
            wait_gather(jb, xg0, semg0)

            @pl.when(jb + 1 < _NCH)
            def _():
                issue_gather(jb + 1, xg1, semg1)

            scale(xg0)
            pltpu.sync_copy(xg0, acc_s.at[cols_v.at[jb]], add=True)
            return carry

        lax.fori_loop(0, (_NCH - 1) // 2, pair, 0)
        plsc.subcore_barrier()
        pltpu.sync_copy(acc_s.at[pl.ds(base, _RPT)],
                        acc_h.at[c, pl.ds(base, _RPT)])

    return k(x, rows_b, cols, al, ar, dv)


def _softplus(z):
    return jnp.maximum(z, 0.0) + jnp.log1p(jnp.exp(-jnp.abs(z)))


def _tc_stage_a(Xs, Xt, deg, W_shr, b_shr, W_src, b_src, W_trg, b_trg,
                Wd1, bd1, Wd2, bd2, att_l, att_r):
    BLK = 1000
    G = (2 * _N) // BLK
    half = G // 2

    def body(xs_ref, xt_ref, deg_ref, wsh_ref, bsh_ref, wsr_ref, bsr_ref,
             wtg_ref, btg_ref, wd1_ref, bd1_ref, wd2_ref, bd2_ref, atl_ref,
             atr_ref, h_ref, al_ref, ar_ref, dinv_ref, disc_ref):
        i = pl.program_id(0)
        x = jnp.where(i < half, xs_ref[...], xt_ref[...])
        shr = jnp.maximum(
            jnp.dot(x, wsh_ref[...], preferred_element_type=jnp.float32)
            + bsh_ref[...], 0.0)
        wown = jnp.where(i < half, wsr_ref[...], wtg_ref[...])
        bown = jnp.where(i < half, bsr_ref[...], btg_ref[...])
        h = jnp.maximum(
            jnp.dot(x, wown, preferred_element_type=jnp.float32) + bown,
            0.0) + shr

        def disc(m):
            z1 = jnp.maximum(
                jnp.dot(m, wd1_ref[...], preferred_element_type=jnp.float32)
                + bd1_ref[...], 0.0)
            return (jnp.dot(z1, wd2_ref[...],
                            preferred_element_type=jnp.float32)
                    + bd2_ref[...])

        sgn = jnp.where(i < half, 1.0, -1.0)
        part = (jnp.sum(_softplus(sgn * disc(shr)))
                + jnp.sum(_softplus(sgn * disc(h))))

        @pl.when(i == 0)
        def _():
            disc_ref[...] = jnp.zeros((1, 1), jnp.float32)

        disc_ref[...] += jnp.reshape(part, (1, 1))
        h_ref[...] = h
        al_ref[...] = jnp.dot(h, atl_ref[...],
                              preferred_element_type=jnp.float32)
        ar_ref[...] = jnp.dot(h, atr_ref[...],
                              preferred_element_type=jnp.float32)
        dinv_ref[...] = lax.rsqrt(deg_ref[:, 0:1] + 1.0)

    full = lambda shape: pl.BlockSpec(shape, lambda i: (0, 0))
    return pl.pallas_call(
        body,
        grid=(G,),
        in_specs=[
            pl.BlockSpec((BLK, _DIM), lambda i: (jnp.minimum(i, half - 1), 0)),
            pl.BlockSpec((BLK, _DIM),
                         lambda i: (jnp.maximum(i - half, 0), 0)),
            pl.BlockSpec((BLK, 16), lambda i: (i, 0)),
            full((_DIM, _H)), full((1, _H)),
            full((_DIM, _H)), full((1, _H)),
            full((_DIM, _H)), full((1, _H)),
            full((_H, 16)), full((1, 16)),
            full((16, 1)), full((1, 1)),
            full((_H, 1)), full((_H, 1)),
        ],
        out_specs=[
            pl.BlockSpec((BLK, _H), lambda i: (i, 0)),
            pl.BlockSpec((BLK, 1), lambda i: (i, 0)),
            pl.BlockSpec((BLK, 1), lambda i: (i, 0)),
            pl.BlockSpec((BLK, 1), lambda i: (i, 0)),
            pl.BlockSpec((1, 1), lambda i: (0, 0)),
        ],
        out_shape=[
            jax.ShapeDtypeStruct((2 * _N, _H), jnp.float32),
            jax.ShapeDtypeStruct((2 * _N, 1), jnp.float32),
            jax.ShapeDtypeStruct((2 * _N, 1), jnp.float32),
            jax.ShapeDtypeStruct((2 * _N, 1), jnp.float32),
            jax.ShapeDtypeStruct((1, 1), jnp.float32),
        ],
    )(Xs, Xt, deg, W_shr, b_shr, W_src, b_src, W_trg, b_trg,
      Wd1, bd1, Wd2, bd2, att_l, att_r)


def _tc_stage_c(acc, h, al, ar, dinv, att_l, att_r):
    BLK = 2000
    G = (2 * _N) // BLK

    def body(acc_ref, h_ref, al_ref, ar_ref, dv_ref, atl_ref, atr_ref,
             x1_ref, al2_ref, ar2_ref):
        dv = dv_ref[...]
        coef = jnp.tanh(al_ref[...] + ar_ref[...]) * dv * dv + _EPS
        x1 = acc_ref[...] + coef * h_ref[...]
        x1_ref[...] = x1
        al2_ref[...] = jnp.dot(x1, atl_ref[...],
                               preferred_element_type=jnp.float32)
        ar2_ref[...] = jnp.dot(x1, atr_ref[...],
                               preferred_element_type=jnp.float32)

    col = lambda: pl.BlockSpec((BLK, 1), lambda i: (i, 0))
    return pl.pallas_call(
        body,
        grid=(G,),
        in_specs=[
            pl.BlockSpec((BLK, _H), lambda i: (i, 0)),
            pl.BlockSpec((BLK, _H), lambda i: (i, 0)),
            col(), col(), col(),
            pl.BlockSpec((_H, 1), lambda i: (0, 0)),
            pl.BlockSpec((_H, 1), lambda i: (0, 0)),
        ],
        out_specs=[
            pl.BlockSpec((BLK, _H), lambda i: (i, 0)),
            col(), col(),
        ],
        out_shape=[
            jax.ShapeDtypeStruct((2 * _N, _H), jnp.float32),
            jax.ShapeDtypeStruct((2 * _N, 1), jnp.float32),
            jax.ShapeDtypeStruct((2 * _N, 1), jnp.float32),
        ],
    )(acc, h, al, ar, dinv, att_l, att_r)


def _tc_stage_d(acc2, x1, h, al2, ar2, dinv, W_pred, b_pred):
    BLK = 2000
    G = (2 * _N) // BLK

    def body(acc_ref, x1_ref, h_ref, al_ref, ar_ref, dv_ref, wp_ref, bp_ref,
             out_ref):
        dv = dv_ref[...]
        coef = jnp.tanh(al_ref[...] + ar_ref[...]) * dv * dv
        y = acc_ref[...] + coef * x1_ref[...] + _EPS * h_ref[...]
        out_ref[...] = (jnp.dot(y, wp_ref[...],
                                preferred_element_type=jnp.float32)
                        + bp_ref[...])

    col = lambda: pl.BlockSpec((BLK, 1), lambda i: (i, 0))
    return pl.pallas_call(
        body,
        grid=(G,),
        in_specs=[
            pl.BlockSpec((BLK, _H), lambda i: (i, 0)),
            pl.BlockSpec((BLK, _H), lambda i: (i, 0)),
            pl.BlockSpec((BLK, _H), lambda i: (i, 0)),
            col(), col(), col(),
            pl.BlockSpec((_H, 2), lambda i: (0, 0)),
            pl.BlockSpec((1, 2), lambda i: (0, 0)),
        ],
        out_specs=pl.BlockSpec((BLK, 2), lambda i: (i, 0)),
        out_shape=jax.ShapeDtypeStruct((2 * _N, 2), jnp.float32),
    )(acc2, x1, h, al2, ar2, dinv, W_pred, b_pred)


def kernel(src, src_edge, trg, trg_edge, W_src, b_src, W_shr, b_shr,
           W_trg, b_trg, Wd1, bd1, Wd2, bd2, att_l, att_r, W_pred, b_pred):
    e_s = src_edge[0]
    e_t = trg_edge[0]
    rows_b = jnp.concatenate(
        [e_s[0], e_t[0] + _N]).reshape(32, _NCH, _CH)
    cols = jnp.concatenate([e_s[1], e_t[1]]).reshape(32, _NCH, _CH)

    deg = _sc_degree(cols)

    h, al1, ar1, dinv, disc_sum = _tc_stage_a(
        src, trg, deg[:, :_N].reshape(2 * _N, 16), W_shr, b_shr.reshape(1, _H),
        W_src, b_src.reshape(1, _H), W_trg, b_trg.reshape(1, _H),
        Wd1, bd1.reshape(1, 16), Wd2, bd2.reshape(1, 1), att_l, att_r)

    al1_t = al1.reshape(2, _N)
    ar1_t = ar1.reshape(2, _N)
    dv_t = dinv.reshape(2, _N)

    acc1 = _sc_msgpass(h, rows_b, cols, al1_t, ar1_t, dv_t)
    x1, al2, ar2 = _tc_stage_c(acc1[:, :_N].reshape(2 * _N, _H), h, al1,
                               ar1, dinv, att_l, att_r)
    acc2 = _sc_msgpass(x1, rows_b, cols, al2.reshape(2, _N),
                       ar2.reshape(2, _N), dv_t)
    out = _tc_stage_d(acc2[:, :_N].reshape(2 * _N, _H), x1, h, al2, ar2,
                      dinv, W_pred, b_pred.reshape(1, 2))

    disc_loss = disc_sum[0, 0] / jnp.float32(_N)
    return out[:_N], out[_N:], disc_loss
